# baseline (speedup 1.0000x reference)
import jax
import jax.numpy as jnp
from jax.experimental import pallas as pl

_BLOCK_ROWS = 2048


def _bcast_kernel(table_ref, out_ref):
    out_ref[:, :] = jnp.broadcast_to(table_ref[:, :], out_ref.shape)


def kernel(ref_tensor, table):
    batch, _ = ref_tensor.shape
    dim = table.shape[1]
    grid = batch // _BLOCK_ROWS
    return pl.pallas_call(
        _bcast_kernel,
        grid=(grid,),
        in_specs=[pl.BlockSpec((1, dim), lambda i: (0, 0))],
        out_specs=pl.BlockSpec((_BLOCK_ROWS, dim), lambda i: (i, 0)),
        out_shape=jax.ShapeDtypeStruct((batch, dim), table.dtype),
    )(table)


# speedup vs baseline: 1.3393x; 1.3393x over previous
import jax
import jax.numpy as jnp
from jax.experimental import pallas as pl
from jax.experimental.pallas import tpu as pltpu

_STAGE_ROWS = 2048


def kernel(ref_tensor, table):
    batch, _ = ref_tensor.shape
    dim = table.shape[1]
    n_copies = batch // _STAGE_ROWS

    def body(table_ref, out_ref, stage, sem):
        stage[:, :] = jnp.broadcast_to(table_ref[:, :], stage.shape)
        copies = [
            pltpu.make_async_copy(
                stage, out_ref.at[pl.ds(i * _STAGE_ROWS, _STAGE_ROWS)], sem
            )
            for i in range(n_copies)
        ]
        for cp in copies:
            cp.start()
        for cp in copies:
            cp.wait()

    return pl.pallas_call(
        body,
        in_specs=[pl.BlockSpec(memory_space=pltpu.VMEM)],
        out_specs=pl.BlockSpec(memory_space=pltpu.MemorySpace.HBM),
        out_shape=jax.ShapeDtypeStruct((batch, dim), table.dtype),
        scratch_shapes=[
            pltpu.VMEM((_STAGE_ROWS, dim), jnp.float32),
            pltpu.SemaphoreType.DMA,
        ],
    )(table)
